# baseline (speedup 1.0000x reference)
import functools

import jax
import jax.numpy as jnp
from jax import lax
from jax.experimental import pallas as pl
from jax.experimental.pallas import tpu as pltpu
from jax.experimental.pallas import tpu_sc as plsc

D = 128
NR = 6
NS = 7
NB = 8
E = 320000
T = 640000

_RA = 2000
_RC = 2000
_RE = 2000


def _swish(v):
    return v * jax.nn.sigmoid(v)


def _pre_body(x_ref, rbf_ref, wkj_ref, bkj_ref, wrbf_ref, out_ref):
    x = x_ref[...]
    xk = _swish(jnp.dot(x, wkj_ref[...], preferred_element_type=jnp.float32)
                + bkj_ref[...])
    rh = jnp.zeros((x.shape[0], D), jnp.float32)
    for r in range(NR):
        rh = rh + rbf_ref[:, r:r + 1] * wrbf_ref[r:r + 1, :]
    out_ref[...] = xk * rh


def _pre(x, rbf, W_kj, b_kj, W_rbf):
    grid = (E // _RA,)
    return pl.pallas_call(
        _pre_body,
        grid=grid,
        in_specs=[
            pl.BlockSpec((_RA, D), lambda i: (i, 0)),
            pl.BlockSpec((_RA, NR), lambda i: (i, 0)),
            pl.BlockSpec((D, D), lambda i: (0, 0)),
            pl.BlockSpec((1, D), lambda i: (0, 0)),
            pl.BlockSpec((NR, D), lambda i: (0, 0)),
        ],
        out_specs=pl.BlockSpec((_RA, D), lambda i: (i, 0)),
        out_shape=jax.ShapeDtypeStruct((E, D), jnp.float32),
    )(x, rbf, W_kj, b_kj.reshape(1, D), W_rbf)


_GW = 128
_T_PAD = 655360


def _gather(table, idx_padded):
    idx2 = idx_padded.reshape(1, _T_PAD)
    mesh = plsc.VectorSubcoreMesh(core_axis_name="c", subcore_axis_name="s")

    @functools.partial(
        pl.kernel,
        out_type=jax.ShapeDtypeStruct((_T_PAD, D), jnp.float32),
        mesh=mesh,
    )
    def k(x_hbm, i_hbm, o_hbm):
        def body(i_vmem, o_vmem):
            pltpu.sync_copy(x_hbm.at[i_vmem.at[0]], o_vmem)

        pltpu.emit_pipeline(
            body,
            grid=(_T_PAD // _GW,),
            in_specs=[pl.BlockSpec((1, _GW), lambda i: (0, i))],
            out_specs=[pl.BlockSpec((_GW, D), lambda i: (i, 0))],
            core_axis_name=("c", "s"),
            dimension_semantics=(pltpu.PARALLEL,),
        )(i_hbm, o_hbm)

    return k(table, idx2)


def _msg_body(g_ref, sbf_ref, wsbf_ref, w2_ref, out_ref):
    sh = jnp.dot(sbf_ref[...], wsbf_ref[...], preferred_element_type=jnp.float32)
    q = jnp.dot(g_ref[...], w2_ref[...], preferred_element_type=jnp.float32)
    acc = jnp.zeros((g_ref.shape[0], D), jnp.float32)
    for i in range(NB):
        acc = acc + sh[:, i:i + 1] * q[:, i * D:(i + 1) * D]
    out_ref[...] = acc


def _msg(gathered, sbf, W_sbf, W2):
    grid = (T // _RC,)
    return pl.pallas_call(
        _msg_body,
        grid=grid,
        in_specs=[
            pl.BlockSpec((_RC, D), lambda i: (i, 0)),
            pl.BlockSpec((_RC, NS * NR), lambda i: (i, 0)),
            pl.BlockSpec((NS * NR, NB), lambda i: (0, 0)),
            pl.BlockSpec((D, NB * D), lambda i: (0, 0)),
        ],
        out_specs=pl.BlockSpec((_RC, D), lambda i: (i, 0)),
        out_shape=jax.ShapeDtypeStruct((T, D), jnp.float32),
    )(gathered, sbf, W_sbf, W2)


_HALF = E // 2
_CH = 16000
_NCH = _HALF // _CH
_SPR = 16128
_TRASH = 16064
_ZR = 252
_SLICE = T // 16
_IT = 4000
_NIT = _SLICE // _IT
_RB = 128
_RING = 64


def _scatter_body(msg_hbm, ji_hbm, zero_hbm, agg_hbm,
                  itile, tbuf, dbuf, rows, zbuf, spmem, sem):
    c = lax.axis_index("c")
    s = lax.axis_index("s")
    half_lo = c * _HALF
    pltpu.sync_copy(zero_hbm, zbuf)

    def drain(dr):
        row = (dr >> 7) & (_RING - 1)
        pltpu.async_copy(msg_hbm.at[tbuf.at[row]], rows, sem).wait()
        pltpu.sync_copy(rows, spmem.at[dbuf.at[row]], add=True)
        return dr + _RB

    @pl.loop(0, _NCH)
    def _chunk(k):
        lo = half_lo + k * _CH

        @pl.loop(0, 4)
        def _z(z):
            pltpu.sync_copy(zbuf, spmem.at[pl.ds(s * (4 * _ZR) + z * _ZR, _ZR)])
        plsc.subcore_barrier()

        def tile_step(j, carry):
            cnt, dr = carry
            tbase = s * _SLICE + j * _IT
            pltpu.sync_copy(ji_hbm.at[pl.ds(tbase, _IT)], itile)

            def vec_step(i, cntv):
                off = i * 16
                d = itile[pl.ds(off, 16)]
                m = (d >= lo) & (d < lo + _CH)
                one = m.astype(jnp.int32)
                pos = cntv + plsc.cumsum(one) - one
                prow = (pos >> 7) & (_RING - 1)
                pcol = pos & 127
                tval = tbase + off + lax.iota(jnp.int32, 16)
                plsc.store_scatter(tbuf, [prow, pcol], tval, mask=m)
                plsc.store_scatter(dbuf, [prow, pcol], d - lo, mask=m)
                return cntv + plsc.all_reduce_population_count(m)

            cntv = jnp.full((16,), cnt, jnp.int32)
            cntv = lax.fori_loop(0, _IT // 16, vec_step, cntv)
            cnt = lax.reduce_max(cntv, axes=(0,))

            dr = lax.while_loop(lambda v: cnt - v >= _RB, drain, dr)
            return cnt, dr

        cnt, dr = lax.fori_loop(0, _NIT, tile_step, (jnp.int32(0), jnp.int32(0)))

        pad_n = (_RB - (cnt & (_RB - 1))) & (_RB - 1)

        def pad_step(jj, _):
            pos = cnt + jj * 16 + lax.iota(jnp.int32, 16)
            m = (jj * 16 + lax.iota(jnp.int32, 16)) < pad_n
            prow = (pos >> 7) & (_RING - 1)
            pcol = pos & 127
            plsc.store_scatter(tbuf, [prow, pcol], jnp.zeros((16,), jnp.int32),
                               mask=m)
            plsc.store_scatter(dbuf, [prow, pcol],
                               jnp.full((16,), _TRASH, jnp.int32), mask=m)
            return 0

        lax.fori_loop(0, _RB // 16, pad_step, 0)
        cnt = cnt + pad_n
        dr = lax.while_loop(lambda v: cnt - v >= _RB, drain, dr)

        plsc.subcore_barrier()
        pltpu.sync_copy(spmem.at[pl.ds(s * (_CH // 16), _CH // 16)],
                        agg_hbm.at[pl.ds(lo + s * (_CH // 16), _CH // 16)])
        plsc.subcore_barrier()


def _scatter_add(msg, ji, zeros):
    mesh = plsc.VectorSubcoreMesh(core_axis_name="c", subcore_axis_name="s")
    k = functools.partial(
        pl.kernel,
        out_type=jax.ShapeDtypeStruct((E, D), jnp.float32),
        mesh=mesh,
        scratch_types=[
            pltpu.VMEM((_IT,), jnp.int32),
            pltpu.VMEM((_RING, _RB), jnp.int32),
            pltpu.VMEM((_RING, _RB), jnp.int32),
            pltpu.VMEM((_RB, D), jnp.float32),
            pltpu.VMEM((_ZR, D), jnp.float32),
            pltpu.VMEM_SHARED((_SPR, D), jnp.float32),
            pltpu.SemaphoreType.DMA,
        ],
    )(_scatter_body)
    return k(msg, ji, zeros)


def _post_body(x_ref, agg_ref, wji_ref, bji_ref,
               w1a_ref, b1a_ref, w1b_ref, b1b_ref,
               wd_ref, bd_ref,
               w2a_ref, b2a_ref, w2b_ref, b2b_ref,
               w3a_ref, b3a_ref, w3b_ref, b3b_ref,
               out_ref):
    x = x_ref[...]

    def mm(v, w_ref, b_ref):
        return jnp.dot(v, w_ref[...], preferred_element_type=jnp.float32) + b_ref[...]

    def resid(h, wa, ba, wb, bb):
        u = _swish(mm(h, wa, ba))
        u = _swish(mm(u, wb, bb))
        return h + u

    h = _swish(mm(x, wji_ref, bji_ref)) + agg_ref[...]
    h = resid(h, w1a_ref, b1a_ref, w1b_ref, b1b_ref)
    h = _swish(mm(h, wd_ref, bd_ref)) + x
    h = resid(h, w2a_ref, b2a_ref, w2b_ref, b2b_ref)
    h = resid(h, w3a_ref, b3a_ref, w3b_ref, b3b_ref)
    out_ref[...] = h


def _post(x, agg, weights):
    grid = (E // _RE,)
    row = pl.BlockSpec((_RE, D), lambda i: (i, 0))
    wspec = pl.BlockSpec((D, D), lambda i: (0, 0))
    bspec = pl.BlockSpec((1, D), lambda i: (0, 0))
    in_specs = [row, row]
    args = [x, agg]
    for (w, b) in weights:
        in_specs += [wspec, bspec]
        args += [w, b.reshape(1, D)]
    return pl.pallas_call(
        _post_body,
        grid=grid,
        in_specs=in_specs,
        out_specs=row,
        out_shape=jax.ShapeDtypeStruct((E, D), jnp.float32),
    )(*args)


def kernel(x, rbf, sbf, edge_idx_kj, edge_idx_ji, W_rbf, W_sbf, W_kj, b_kj,
           W_ji, b_ji, W_bil, W_r1a, b_r1a, W_r1b, b_r1b, W_d, b_d,
           W_r2a, b_r2a, W_r2b, b_r2b, W_r3a, b_r3a, W_r3b, b_r3b):
    x_kj = _pre(x, rbf, W_kj, b_kj, W_rbf)
    idx_p = jnp.concatenate(
        [edge_idx_kj, jnp.zeros((_T_PAD - T,), jnp.int32)])
    gathered = _gather(x_kj, idx_p)
    W2 = jnp.transpose(W_bil, (2, 1, 0)).reshape(D, NB * D)
    msg = _msg(gathered, sbf, W_sbf, W2)
    agg = jnp.zeros((E, D), jnp.float32).at[edge_idx_ji].add(msg)
    weights = [(W_ji, b_ji), (W_r1a, b_r1a), (W_r1b, b_r1b), (W_d, b_d),
               (W_r2a, b_r2a), (W_r2b, b_r2b), (W_r3a, b_r3a), (W_r3b, b_r3b)]
    return _post(x, agg, weights)
